# baseline (device time: 17430 ns/iter reference)
import jax
import jax.numpy as jnp
from jax import lax
from jax.experimental import pallas as pl
from jax.experimental.pallas import tpu as pltpu

N_DEV = 8
M = 512
D = 512
BLK = M // N_DEV


def kernel(partial, resid, gamma):
    partial2 = partial.reshape(M, D)
    gamma2 = gamma.reshape(1, D)

    def body(
        p_ref,
        r_ref,
        g_ref,
        out_ref,
        send_buf,
        gather_buf,
        out_bf16,
        bcast_buf,
        send_sems1, recv_sems1, send_sems2, recv_sems2,
    ):
        my = lax.axis_index("i")

        barrier = pltpu.get_barrier_semaphore()
        for d in range(1, N_DEV):
            peer = lax.rem(my + d, N_DEV)
            pl.semaphore_signal(
                barrier, inc=1,
                device_id=(peer,), device_id_type=pl.DeviceIdType.MESH,
            )
        pl.semaphore_wait(barrier, N_DEV - 1)

        send_buf[...] = p_ref[...].astype(jnp.bfloat16)

        phase1 = []
        for d in range(1, N_DEV):
            peer = lax.rem(my + d, N_DEV)
            rdma = pltpu.make_async_remote_copy(
                src_ref=send_buf.at[pl.ds(peer * BLK, BLK), :],
                dst_ref=gather_buf.at[d - 1],
                send_sem=send_sems1.at[d - 1],
                recv_sem=recv_sems1.at[d - 1],
                device_id=(peer,),
                device_id_type=pl.DeviceIdType.MESH,
            )
            rdma.start()
            phase1.append(rdma)

        row0 = my * BLK
        acc = p_ref[pl.ds(row0, BLK), :] + r_ref[pl.ds(row0, BLK), :]

        for d in range(1, N_DEV):
            phase1[d - 1].wait_recv()
            acc = acc + gather_buf[d - 1].astype(jnp.float32)

        rms = jnp.sqrt(jnp.mean(acc * acc, axis=-1, keepdims=True) + 1e-6)
        blk = (acc / rms) * g_ref[0, :][None, :]
        out_ref[pl.ds(row0, BLK), :] = blk
        out_bf16[...] = blk.astype(jnp.bfloat16)

        phase2 = []
        for d in range(1, N_DEV):
            peer = lax.rem(my + d, N_DEV)
            rdma = pltpu.make_async_remote_copy(
                src_ref=out_bf16,
                dst_ref=bcast_buf.at[d - 1],
                send_sem=send_sems2.at[d - 1],
                recv_sem=recv_sems2.at[d - 1],
                device_id=(peer,),
                device_id_type=pl.DeviceIdType.MESH,
            )
            rdma.start()
            phase2.append(rdma)

        for d in range(1, N_DEV):
            phase2[d - 1].wait_recv()
            src_dev = lax.rem(my - d + N_DEV, N_DEV)
            out_ref[pl.ds(src_dev * BLK, BLK), :] = (
                bcast_buf[d - 1].astype(jnp.float32)
            )

        for d in range(1, N_DEV):
            phase1[d - 1].wait_send()
            phase2[d - 1].wait_send()

    return pl.pallas_call(
        body,
        out_shape=jax.ShapeDtypeStruct((M, D), jnp.float32),
        in_specs=[
            pl.BlockSpec(memory_space=pltpu.VMEM),
            pl.BlockSpec(memory_space=pltpu.VMEM),
            pl.BlockSpec(memory_space=pltpu.VMEM),
        ],
        out_specs=pl.BlockSpec(memory_space=pltpu.VMEM),
        scratch_shapes=[
            pltpu.VMEM((M, D), jnp.bfloat16),
            pltpu.VMEM((N_DEV - 1, BLK, D), jnp.bfloat16),
            pltpu.VMEM((BLK, D), jnp.bfloat16),
            pltpu.VMEM((N_DEV - 1, BLK, D), jnp.bfloat16),
            pltpu.SemaphoreType.DMA((N_DEV - 1,)),
            pltpu.SemaphoreType.DMA((N_DEV - 1,)),
            pltpu.SemaphoreType.DMA((N_DEV - 1,)),
            pltpu.SemaphoreType.DMA((N_DEV - 1,)),
        ],
        compiler_params=pltpu.CompilerParams(collective_id=0),
    )(partial2, resid, gamma2)
